# baseline (speedup 1.0000x reference)
import jax
import jax.numpy as jnp
from jax.experimental import pallas as pl
from jax.experimental.pallas import tpu as pltpu

S_BLK = 256


def _body(x_ref, pos_ref, out_ref, *, batch, d_model):
    pos = pos_ref[...]
    for b in range(batch):
        sl = slice(b * d_model, (b + 1) * d_model)
        out_ref[:, sl] = x_ref[:, sl] + pos


def kernel(x, pos_table):
    seq_len, batch, d_model = x.shape
    x2 = x.reshape(seq_len, batch * d_model)
    grid = (seq_len // S_BLK,)
    import functools
    out2 = pl.pallas_call(
        functools.partial(_body, batch=batch, d_model=d_model),
        grid=grid,
        in_specs=[
            pl.BlockSpec((S_BLK, batch * d_model), lambda i: (i, 0)),
            pl.BlockSpec((S_BLK, d_model), lambda i: (i, 0)),
        ],
        out_specs=pl.BlockSpec((S_BLK, batch * d_model), lambda i: (i, 0)),
        out_shape=jax.ShapeDtypeStruct((seq_len, batch * d_model), x.dtype),
        compiler_params=pltpu.CompilerParams(
            dimension_semantics=("arbitrary",),
        ),
    )(x2, pos_table[:seq_len])
    return out2.reshape(seq_len, batch, d_model)


# speedup vs baseline: 3.9137x; 3.9137x over previous
import jax
import jax.numpy as jnp
from jax.experimental import pallas as pl
from jax.experimental.pallas import tpu as pltpu

S_BLK = 128


def _body(x_ref, pos_ref, out_ref):
    pos = pos_ref[...]
    for b in range(x_ref.shape[1]):
        out_ref[:, b, :] = x_ref[:, b, :] + pos


def kernel(x, pos_table):
    seq_len, batch, d_model = x.shape
    grid = (seq_len // S_BLK,)
    return pl.pallas_call(
        _body,
        grid=grid,
        in_specs=[
            pl.BlockSpec((S_BLK, batch, d_model), lambda i: (i, 0, 0)),
            pl.BlockSpec((S_BLK, d_model), lambda i: (i, 0)),
        ],
        out_specs=pl.BlockSpec((S_BLK, batch, d_model), lambda i: (i, 0, 0)),
        out_shape=jax.ShapeDtypeStruct((seq_len, batch, d_model), x.dtype),
        compiler_params=pltpu.CompilerParams(
            dimension_semantics=("arbitrary",),
        ),
    )(x, pos_table[:seq_len])


# speedup vs baseline: 4.2262x; 1.0798x over previous
import jax
import jax.numpy as jnp
from jax.experimental import pallas as pl
from jax.experimental.pallas import tpu as pltpu

S_BLK = 256


def _body(x_ref, pos_ref, out_ref):
    pos = pos_ref[...]
    for b in range(x_ref.shape[1]):
        out_ref[:, b, :] = x_ref[:, b, :] + pos


def kernel(x, pos_table):
    seq_len, batch, d_model = x.shape
    grid = (seq_len // S_BLK,)
    return pl.pallas_call(
        _body,
        grid=grid,
        in_specs=[
            pl.BlockSpec((S_BLK, batch, d_model), lambda i: (i, 0, 0)),
            pl.BlockSpec((S_BLK, d_model), lambda i: (i, 0)),
        ],
        out_specs=pl.BlockSpec((S_BLK, batch, d_model), lambda i: (i, 0, 0)),
        out_shape=jax.ShapeDtypeStruct((seq_len, batch, d_model), x.dtype),
        compiler_params=pltpu.CompilerParams(
            dimension_semantics=("arbitrary",),
        ),
    )(x, pos_table[:seq_len])


# speedup vs baseline: 4.2866x; 1.0143x over previous
import jax
import jax.numpy as jnp
from jax.experimental import pallas as pl
from jax.experimental.pallas import tpu as pltpu

S_BLK = 512


def _body(x_ref, pos_ref, out_ref):
    pos = pos_ref[...]
    for b in range(x_ref.shape[1]):
        out_ref[:, b, :] = x_ref[:, b, :] + pos


def kernel(x, pos_table):
    seq_len, batch, d_model = x.shape
    grid = (seq_len // S_BLK,)
    return pl.pallas_call(
        _body,
        grid=grid,
        in_specs=[
            pl.BlockSpec((S_BLK, batch, d_model), lambda i: (i, 0, 0)),
            pl.BlockSpec((S_BLK, d_model), lambda i: (i, 0)),
        ],
        out_specs=pl.BlockSpec((S_BLK, batch, d_model), lambda i: (i, 0, 0)),
        out_shape=jax.ShapeDtypeStruct((seq_len, batch, d_model), x.dtype),
        compiler_params=pltpu.CompilerParams(
            dimension_semantics=("arbitrary",),
        ),
    )(x, pos_table[:seq_len])
